# baseline (speedup 1.0000x reference)
import functools

import jax
import jax.numpy as jnp
from jax import lax
from jax.experimental import pallas as pl
from jax.experimental.pallas import tpu as pltpu
from jax.experimental.pallas import tpu_sc as plsc

B, A, F, D = 64, 512, 128, 16
LANES = 16
NCHUNKS_F = F // LANES

NC, NS = 2, 16
NW = NC * NS
MOLS_PER_W = B // NW
ACHUNK = 128
NACH = A // ACHUNK


def _dyn_gather(vec, idx):
    dn = lax.GatherDimensionNumbers(
        offset_dims=(), collapsed_slice_dims=(0,), start_index_map=(0,))
    return lax.gather(vec, idx[:, None], dn, (1,),
                      mode=lax.GatherScatterMode.PROMISE_IN_BOUNDS)


def _graph_pool_body(atoms_hbm, edges_hbm, out_hbm, atoms_v, edges_v, out_v, sem):
    wid = lax.axis_index("s") * NC + lax.axis_index("c")

    lanes = lax.broadcasted_iota(jnp.int32, (LANES,), 0)
    cbases = [lanes + c * LANES for c in range(NCHUNKS_F)]
    dconsts = [jnp.full((LANES,), d, jnp.int32) for d in range(D)]

    for m in range(MOLS_PER_W):
        b = wid * MOLS_PER_W + m
        pltpu.sync_copy(atoms_hbm.at[b], atoms_v)
        pltpu.sync_copy(edges_hbm.at[b], edges_v)

        for ch in range(NACH):
            def atom_body(a, carry, ch=ch):
                accs = [atoms_v[ch * ACHUNK + a, pl.ds(c * LANES, LANES)]
                        for c in range(NCHUNKS_F)]
                ev = edges_v[ch * ACHUNK + a, pl.ds(0, D)]
                for d in range(D):
                    rowv = _dyn_gather(ev, dconsts[d])
                    for c in range(NCHUNKS_F):
                        g = plsc.load_gather(atoms_v, [rowv, cbases[c]])
                        accs[c] = jnp.maximum(accs[c], g)
                for c in range(NCHUNKS_F):
                    out_v[a, pl.ds(c * LANES, LANES)] = accs[c]
                return carry

            lax.fori_loop(0, ACHUNK, atom_body, 0)
            pltpu.sync_copy(out_v, out_hbm.at[b, pl.ds(ch * ACHUNK, ACHUNK)])


_graph_pool = pl.kernel(
    _graph_pool_body,
    out_type=jax.ShapeDtypeStruct((B, A, F), jnp.float32),
    mesh=plsc.VectorSubcoreMesh(core_axis_name="c", subcore_axis_name="s"),
    scratch_types=[
        pltpu.VMEM((A, F), jnp.float32),
        pltpu.VMEM((A, D), jnp.int32),
        pltpu.VMEM((ACHUNK, F), jnp.float32),
        pltpu.SemaphoreType.DMA,
    ],
    compiler_params=pltpu.CompilerParams(
        use_tc_tiling_on_sc=False, needs_layout_passes=False),
)


def kernel(atoms, edges):
    return _graph_pool(atoms, edges.astype(jnp.int32))


# speedup vs baseline: 1.1515x; 1.1515x over previous
import functools

import jax
import jax.numpy as jnp
from jax import lax
from jax.experimental import pallas as pl
from jax.experimental.pallas import tpu as pltpu
from jax.experimental.pallas import tpu_sc as plsc

B, A, F, D = 64, 512, 128, 16
LANES = 16
NCHUNKS_F = F // LANES

NC, NS = 2, 16
NW = NC * NS
MOLS_PER_W = B // NW
ACHUNK = 128
NACH = A // ACHUNK


def _dyn_gather(vec, idx):
    dn = lax.GatherDimensionNumbers(
        offset_dims=(), collapsed_slice_dims=(0,), start_index_map=(0,))
    return lax.gather(vec, idx[:, None], dn, (1,),
                      mode=lax.GatherScatterMode.PROMISE_IN_BOUNDS)


def _graph_pool_body(atoms_hbm, edges_hbm, out_hbm, atoms_v, edges_v, out_v, sem):
    wid = lax.axis_index("s") * NC + lax.axis_index("c")

    lanes = lax.broadcasted_iota(jnp.int32, (LANES,), 0)
    cbases = [lanes + c * LANES for c in range(NCHUNKS_F)]
    dconsts = [jnp.full((LANES,), d, jnp.int32) for d in range(D)]

    for m in range(MOLS_PER_W):
        b = wid * MOLS_PER_W + m
        pltpu.sync_copy(atoms_hbm.at[b], atoms_v)
        pltpu.sync_copy(edges_hbm.at[b], edges_v)

        for ch in range(NACH):
            def atom_body(a, ch=ch):
                accs = [atoms_v[ch * ACHUNK + a, pl.ds(c * LANES, LANES)]
                        for c in range(NCHUNKS_F)]
                ev = edges_v[ch * ACHUNK + a, pl.ds(0, D)]
                for d in range(D):
                    row = ev[d]
                    for c in range(NCHUNKS_F):
                        accs[c] = jnp.maximum(
                            accs[c], atoms_v[row, pl.ds(c * LANES, LANES)])
                for c in range(NCHUNKS_F):
                    out_v[a, pl.ds(c * LANES, LANES)] = accs[c]

            plsc.parallel_loop(0, ACHUNK, unroll=2)(atom_body)
            pltpu.sync_copy(out_v, out_hbm.at[b, pl.ds(ch * ACHUNK, ACHUNK)])


_graph_pool = pl.kernel(
    _graph_pool_body,
    out_type=jax.ShapeDtypeStruct((B, A, F), jnp.float32),
    mesh=plsc.VectorSubcoreMesh(core_axis_name="c", subcore_axis_name="s"),
    scratch_types=[
        pltpu.VMEM((A, F), jnp.float32),
        pltpu.VMEM((A, D), jnp.int32),
        pltpu.VMEM((ACHUNK, F), jnp.float32),
        pltpu.SemaphoreType.DMA,
    ],
    compiler_params=pltpu.CompilerParams(
        use_tc_tiling_on_sc=False, needs_layout_passes=False),
)


def kernel(atoms, edges):
    return _graph_pool(atoms, edges.astype(jnp.int32))


# speedup vs baseline: 1.2707x; 1.1035x over previous
import functools

import jax
import jax.numpy as jnp
from jax import lax
from jax.experimental import pallas as pl
from jax.experimental.pallas import tpu as pltpu
from jax.experimental.pallas import tpu_sc as plsc

B, A, F, D = 64, 512, 128, 16
LANES = 16
NCHUNKS_F = F // LANES

NC, NS = 2, 16
NW = NC * NS
MOLS_PER_W = B // NW
ACHUNK = 128
NACH = A // ACHUNK


def _dyn_gather(vec, idx):
    dn = lax.GatherDimensionNumbers(
        offset_dims=(), collapsed_slice_dims=(0,), start_index_map=(0,))
    return lax.gather(vec, idx[:, None], dn, (1,),
                      mode=lax.GatherScatterMode.PROMISE_IN_BOUNDS)


def _graph_pool_body(atoms_hbm, edges_hbm, out_hbm, atoms_v, edges_v, out_v, sem):
    wid = lax.axis_index("s") * NC + lax.axis_index("c")

    lanes = lax.broadcasted_iota(jnp.int32, (LANES,), 0)
    cbases = [lanes + c * LANES for c in range(NCHUNKS_F)]
    dconsts = [jnp.full((LANES,), d, jnp.int32) for d in range(D)]

    for m in range(MOLS_PER_W):
        b = wid * MOLS_PER_W + m
        pltpu.sync_copy(atoms_hbm.at[b], atoms_v)
        pltpu.sync_copy(edges_hbm.at[b], edges_v)

        for ch in range(NACH):
            def atom_body(a, ch=ch):
                accs = [atoms_v[ch * ACHUNK + a, pl.ds(c * LANES, LANES)]
                        for c in range(NCHUNKS_F)]
                ev = edges_v[ch * ACHUNK + a, pl.ds(0, D)]
                for d in range(D):
                    rowv = _dyn_gather(ev, dconsts[d])
                    for c in range(NCHUNKS_F):
                        g = plsc.load_gather(atoms_v, [rowv, cbases[c]])
                        accs[c] = jnp.maximum(accs[c], g)
                for c in range(NCHUNKS_F):
                    out_v[a, pl.ds(c * LANES, LANES)] = accs[c]

            plsc.parallel_loop(0, ACHUNK)(atom_body)
            pltpu.sync_copy(out_v, out_hbm.at[b, pl.ds(ch * ACHUNK, ACHUNK)])


_graph_pool = pl.kernel(
    _graph_pool_body,
    out_type=jax.ShapeDtypeStruct((B, A, F), jnp.float32),
    mesh=plsc.VectorSubcoreMesh(core_axis_name="c", subcore_axis_name="s"),
    scratch_types=[
        pltpu.VMEM((A, F), jnp.float32),
        pltpu.VMEM((A, D), jnp.int32),
        pltpu.VMEM((ACHUNK, F), jnp.float32),
        pltpu.SemaphoreType.DMA,
    ],
    compiler_params=pltpu.CompilerParams(
        use_tc_tiling_on_sc=False, needs_layout_passes=False),
)


def kernel(atoms, edges):
    return _graph_pool(atoms, edges.astype(jnp.int32))
